# Initial kernel scaffold; baseline (speedup 1.0000x reference)
import jax
import jax.numpy as jnp
from jax.experimental import pallas as pl


def kernel(x, experts_weights, experts_bias, gate_w, gate_b):
    raise NotImplementedError("write your pallas kernel here")


# speedup vs baseline: 2.3358x; 2.3358x over previous
import functools

import jax
import jax.numpy as jnp
from jax.experimental import pallas as pl


def _gating_body(gwT_ref, xT_ref, gb_ref, cT_ref):
    E, B = cT_ref.shape
    logits = (
        jnp.dot(gwT_ref[...], xT_ref[...], preferred_element_type=jnp.float32)
        + gb_ref[...]
    )
    m = jnp.max(logits, axis=0, keepdims=True)
    p = jnp.exp(logits - m)
    g = p / jnp.sum(p, axis=0, keepdims=True)

    r_iota = jax.lax.broadcasted_iota(jnp.int32, (E, B), 0)
    m1 = jnp.max(g, axis=0, keepdims=True)
    idx1 = jnp.min(jnp.where(g == m1, r_iota, E), axis=0, keepdims=True)
    oh1 = r_iota == idx1
    gm = jnp.where(oh1, -1.0, g)
    m2 = jnp.max(gm, axis=0, keepdims=True)
    idx2 = jnp.min(jnp.where(gm == m2, r_iota, E), axis=0, keepdims=True)
    oh2 = r_iota == idx2
    cT_ref[...] = jnp.where(oh1 | oh2, g, 0.0)


def _moe_body(x_ref, cT_ref, w_ref, b_ref, out_ref):
    e = pl.program_id(0)
    B = x_ref.shape[0]
    E = cT_ref.shape[0]

    @pl.when(e == 0)
    def _init():
        out_ref[...] = jnp.zeros_like(out_ref)

    contrib = (
        jnp.dot(x_ref[...], w_ref[0], preferred_element_type=jnp.float32)
        + b_ref[0]
    )
    r_iota = jax.lax.broadcasted_iota(jnp.int32, (E, B), 0)
    crow = jnp.sum(jnp.where(r_iota == e, cT_ref[...], 0.0), axis=0, keepdims=True)
    ri = jax.lax.broadcasted_iota(jnp.int32, (B, B), 0)
    ci = jax.lax.broadcasted_iota(jnp.int32, (B, B), 1)
    diag = jnp.where(ri == ci, jnp.broadcast_to(crow, (B, B)), 0.0)
    out_ref[...] += jnp.dot(diag, contrib, preferred_element_type=jnp.float32)


@functools.partial(jax.jit, static_argnames=("interpret",))
def kernel(x, experts_weights, experts_bias, gate_w, gate_b, interpret=False):
    B, D_in = x.shape
    E, _, D_out = experts_weights.shape

    cT = pl.pallas_call(
        _gating_body,
        out_shape=jax.ShapeDtypeStruct((E, B), jnp.float32),
        interpret=interpret,
    )(gate_w.T, x.T, gate_b.reshape(E, 1))

    out = pl.pallas_call(
        _moe_body,
        grid=(E,),
        in_specs=[
            pl.BlockSpec((B, D_in), lambda e: (0, 0)),
            pl.BlockSpec((E, B), lambda e: (0, 0)),
            pl.BlockSpec((1, D_in, D_out), lambda e: (e, 0, 0)),
            pl.BlockSpec((1, 1, D_out), lambda e: (e, 0, 0)),
        ],
        out_specs=pl.BlockSpec((B, D_out), lambda e: (0, 0)),
        out_shape=jax.ShapeDtypeStruct((B, D_out), jnp.float32),
        interpret=interpret,
    )(x, cT, experts_weights, experts_bias.reshape(E, 1, D_out))
    return out


# speedup vs baseline: 2.9896x; 1.2799x over previous
import functools

import jax
import jax.numpy as jnp
from jax.experimental import pallas as pl


def _gating_body(gwT_ref, xT_ref, gb_ref, cT_ref):
    E, B = cT_ref.shape
    logits = (
        jnp.dot(gwT_ref[...], xT_ref[...], preferred_element_type=jnp.float32)
        + gb_ref[...]
    )
    m = jnp.max(logits, axis=0, keepdims=True)
    p = jnp.exp(logits - m)
    g = p / jnp.sum(p, axis=0, keepdims=True)

    r_iota = jax.lax.broadcasted_iota(jnp.int32, (E, B), 0)
    m1 = jnp.max(g, axis=0, keepdims=True)
    idx1 = jnp.min(jnp.where(g == m1, r_iota, E), axis=0, keepdims=True)
    oh1 = r_iota == idx1
    gm = jnp.where(oh1, -1.0, g)
    m2 = jnp.max(gm, axis=0, keepdims=True)
    idx2 = jnp.min(jnp.where(gm == m2, r_iota, E), axis=0, keepdims=True)
    oh2 = r_iota == idx2
    cT_ref[...] = jnp.where(oh1 | oh2, g, 0.0)


def _moe_body(x_ref, cT_ref, w_ref, b_ref, out_ref, *, eb):
    step = pl.program_id(0)
    B = x_ref.shape[0]
    E = cT_ref.shape[0]

    @pl.when(step == 0)
    def _init():
        out_ref[...] = jnp.zeros_like(out_ref)

    acc = out_ref[...]
    ri = jax.lax.broadcasted_iota(jnp.int32, (B, B), 0)
    ci = jax.lax.broadcasted_iota(jnp.int32, (B, B), 1)
    r_iota = jax.lax.broadcasted_iota(jnp.int32, (E, B), 0)
    for j in range(eb):
        e = step * eb + j
        contrib = (
            jnp.dot(x_ref[...], w_ref[j], preferred_element_type=jnp.float32)
            + b_ref[j]
        )
        crow = jnp.sum(jnp.where(r_iota == e, cT_ref[...], 0.0), axis=0, keepdims=True)
        diag = jnp.where(ri == ci, jnp.broadcast_to(crow, (B, B)), 0.0)
        acc = acc + jnp.dot(diag, contrib, preferred_element_type=jnp.float32)
    out_ref[...] = acc


@functools.partial(jax.jit, static_argnames=("interpret",))
def kernel(x, experts_weights, experts_bias, gate_w, gate_b, interpret=False):
    B, D_in = x.shape
    E, _, D_out = experts_weights.shape

    cT = pl.pallas_call(
        _gating_body,
        out_shape=jax.ShapeDtypeStruct((E, B), jnp.float32),
        interpret=interpret,
    )(gate_w.T, x.T, gate_b.reshape(E, 1))

    EB = 2
    out = pl.pallas_call(
        functools.partial(_moe_body, eb=EB),
        grid=(E // EB,),
        in_specs=[
            pl.BlockSpec((B, D_in), lambda e: (0, 0)),
            pl.BlockSpec((E, B), lambda e: (0, 0)),
            pl.BlockSpec((EB, D_in, D_out), lambda e: (e, 0, 0)),
            pl.BlockSpec((EB, 1, D_out), lambda e: (e, 0, 0)),
        ],
        out_specs=pl.BlockSpec((B, D_out), lambda e: (0, 0)),
        out_shape=jax.ShapeDtypeStruct((B, D_out), jnp.float32),
        interpret=interpret,
    )(x, cT, experts_weights, experts_bias.reshape(E, 1, D_out))
    return out


# speedup vs baseline: 3.3964x; 1.1361x over previous
import functools

import jax
import jax.numpy as jnp
from jax.experimental import pallas as pl


def _gating_body(gwT_ref, xT_ref, gb_ref, cT_ref):
    E, B = cT_ref.shape
    logits = (
        jnp.dot(gwT_ref[...], xT_ref[...], preferred_element_type=jnp.float32)
        + gb_ref[...]
    )
    m = jnp.max(logits, axis=0, keepdims=True)
    p = jnp.exp(logits - m)
    g = p / jnp.sum(p, axis=0, keepdims=True)

    r_iota = jax.lax.broadcasted_iota(jnp.int32, (E, B), 0)
    m1 = jnp.max(g, axis=0, keepdims=True)
    idx1 = jnp.min(jnp.where(g == m1, r_iota, E), axis=0, keepdims=True)
    oh1 = r_iota == idx1
    gm = jnp.where(oh1, -1.0, g)
    m2 = jnp.max(gm, axis=0, keepdims=True)
    idx2 = jnp.min(jnp.where(gm == m2, r_iota, E), axis=0, keepdims=True)
    oh2 = r_iota == idx2
    cT_ref[...] = jnp.where(oh1 | oh2, g, 0.0)


def _moe_body(x_ref, cT_ref, w_ref, b_ref, out_ref, *, eb):
    step = pl.program_id(0)
    B = x_ref.shape[0]
    E = cT_ref.shape[0]

    @pl.when(step == 0)
    def _init():
        out_ref[...] = jnp.zeros_like(out_ref)

    acc = out_ref[...]
    ri = jax.lax.broadcasted_iota(jnp.int32, (B, B), 0)
    ci = jax.lax.broadcasted_iota(jnp.int32, (B, B), 1)
    r_iota = jax.lax.broadcasted_iota(jnp.int32, (E, B), 0)
    for j in range(eb):
        e = step * eb + j
        contrib = (
            jnp.dot(x_ref[...], w_ref[j], preferred_element_type=jnp.float32)
            + b_ref[j]
        )
        crow = jnp.sum(jnp.where(r_iota == e, cT_ref[...], 0.0), axis=0, keepdims=True)
        diag = jnp.where(ri == ci, jnp.broadcast_to(crow, (B, B)), 0.0)
        acc = acc + jnp.dot(diag, contrib, preferred_element_type=jnp.float32)
    out_ref[...] = acc


@functools.partial(jax.jit, static_argnames=("interpret",))
def kernel(x, experts_weights, experts_bias, gate_w, gate_b, interpret=False):
    B, D_in = x.shape
    E, _, D_out = experts_weights.shape

    cT = pl.pallas_call(
        _gating_body,
        out_shape=jax.ShapeDtypeStruct((E, B), jnp.float32),
        interpret=interpret,
    )(gate_w.T, x.T, gate_b.reshape(E, 1))

    EB = 4
    out = pl.pallas_call(
        functools.partial(_moe_body, eb=EB),
        grid=(E // EB,),
        in_specs=[
            pl.BlockSpec((B, D_in), lambda e: (0, 0)),
            pl.BlockSpec((E, B), lambda e: (0, 0)),
            pl.BlockSpec((EB, D_in, D_out), lambda e: (e, 0, 0)),
            pl.BlockSpec((EB, 1, D_out), lambda e: (e, 0, 0)),
        ],
        out_specs=pl.BlockSpec((B, D_out), lambda e: (0, 0)),
        out_shape=jax.ShapeDtypeStruct((B, D_out), jnp.float32),
        interpret=interpret,
    )(x, cT, experts_weights, experts_bias.reshape(E, 1, D_out))
    return out
